# baseline (speedup 1.0000x reference)
import jax
import jax.numpy as jnp
from jax import lax
from jax.experimental import pallas as pl
from jax.experimental.pallas import tpu as pltpu
from jax.experimental.pallas import tpu_sc as plsc

B = 4096
L = 50
H = 100
OUT = 128

NC = 2
NS = 16
NW = NC * NS
BPW = B // NW

_OFFS = (0, 16, 32, 48, 64, 80, 84)
_SLOT = [(k // 16, k % 16) if k < 48 else (3, k - 34) for k in range(L)]
ROWS_BUF = 8 * L + 8


def _sc_pool_body(idx_hbm, table_hbm, out_hbm, idx_v, rows_v, acc_v, sems):
    wid = lax.axis_index("s")

    pltpu.sync_copy(idx_hbm.at[pl.ds(wid * (BPW * L), BPW * L)], idx_v)

    def enqueue(cc):
        buf = lax.bitwise_and(cc, 1)
        iv = [idx_v[pl.ds(cc * L + o, 16)] for o in (0, 16, 32, 34)]
        for k in range(L):
            j, lane = _SLOT[k]
            i = iv[j][lane]
            off = pl.multiple_of(lax.bitwise_and(i, -8), 8)
            d = 8 * k + 7 - (i - off)
            pltpu.async_copy(
                table_hbm.at[pl.ds(off, 8)],
                rows_v.at[buf, pl.ds(d, 8)],
                sems.at[buf],
            )

    enqueue(0)

    def chunk_body(c, _):
        @pl.when(c + 1 < BPW)
        def _pre():
            enqueue(c + 1)

        buf = lax.bitwise_and(c, 1)
        pltpu.make_async_copy(
            table_hbm.at[pl.ds(0, 8 * L)],
            rows_v.at[buf, pl.ds(0, 8 * L)],
            sems.at[buf],
        ).wait()

        accs = [rows_v[buf, 7, pl.ds(_OFFS[s], 16)] for s in range(len(_OFFS))]
        for w in range(1, L):
            for s in range(len(_OFFS)):
                accs[s] = accs[s] + rows_v[buf, 8 * w + 7, pl.ds(_OFFS[s], 16)]
        for s in range(len(_OFFS)):
            acc_v[pl.ds(c * H + _OFFS[s], 16)] = accs[s]
        return 0

    lax.fori_loop(0, BPW, chunk_body, 0)

    pltpu.sync_copy(acc_v, out_hbm.at[pl.ds(wid * (BPW * H), BPW * H)])


def _sc_pool_half(idx_flat_half, table):
    mesh = plsc.VectorSubcoreMesh(
        core_axis_name="c", subcore_axis_name="s", num_cores=1
    )
    return pl.kernel(
        _sc_pool_body,
        out_type=jax.ShapeDtypeStruct((B * H // 2,), jnp.float32),
        mesh=mesh,
        scratch_types=[
            pltpu.VMEM((BPW * L,), jnp.int32),
            pltpu.VMEM((2, ROWS_BUF, H), jnp.float32),
            pltpu.VMEM((BPW * H,), jnp.float32),
            pltpu.SemaphoreType.DMA((2,)),
        ],
    )(idx_flat_half, table)


def _tc_matmul_body(x_ref, w_ref, b_ref, o_ref):
    x = x_ref[...] * (1.0 / L)
    o_ref[...] = (
        lax.dot_general(
            x, w_ref[...], (((1,), (1,)), ((), ())),
            preferred_element_type=jnp.float32,
        )
        + b_ref[...]
    )


def _tc_matmul(pooled, W, b2d):
    return pl.pallas_call(
        _tc_matmul_body,
        out_shape=jax.ShapeDtypeStruct((B, OUT), jnp.float32),
    )(pooled, W, b2d)


def kernel(keyword_tensor_list, word_embed, W, b):
    idx_flat = keyword_tensor_list.astype(jnp.int32).reshape(B * L)
    half = B * L // 2
    p0 = _sc_pool_half(idx_flat[:half], word_embed)
    p1 = _sc_pool_half(idx_flat[half:], word_embed)
    pooled = jnp.concatenate([p0, p1]).reshape(B, H)
    return _tc_matmul(pooled, W, b.reshape(1, OUT))


# speedup vs baseline: 1.3403x; 1.3403x over previous
import jax
import jax.numpy as jnp
from jax import lax
from jax.experimental import pallas as pl
from jax.experimental.pallas import tpu as pltpu
from jax.experimental.pallas import tpu_sc as plsc

B = 4096
L = 50
H = 100
HP = 128
OUT = 128
V = 1000000

NC = 2
NS = 16
NW = NC * NS
BPW = B // NW
CR = 2
N_CHUNKS = BPW // CR
CI = CR * L
CIP = CI + 4

_OFFS = (0, 16, 32, 48, 64, 80, 84)


def _pad_body(x_ref, o_ref):
    o_ref[...] = jnp.pad(x_ref[...], ((0, 0), (0, HP - H)))


def _pad_table(table):
    blk = 16384
    return pl.pallas_call(
        _pad_body,
        grid=(V // blk,),
        in_specs=[pl.BlockSpec((blk, H), lambda i: (i, 0))],
        out_specs=pl.BlockSpec((blk, HP), lambda i: (i, 0)),
        out_shape=jax.ShapeDtypeStruct((V, HP), jnp.float32),
    )(table)


def _sc_pool_body(idx_hbm, table_hbm, out_hbm, idx_v, rows_v, acc_v, sems):
    wid = lax.axis_index("s") * NC + lax.axis_index("c")

    pltpu.sync_copy(idx_hbm.at[pl.ds(wid * (BPW * L), BPW * L)], idx_v)

    def enqueue(cc):
        buf = lax.bitwise_and(cc, 1)
        a8 = pl.multiple_of(lax.bitwise_and(cc * CI, -8), 8)
        pltpu.async_copy(
            table_hbm.at[idx_v.at[pl.ds(a8, CIP)]],
            rows_v.at[buf],
            sems.at[buf],
        )

    enqueue(0)

    def chunk_body(c, _):
        @pl.when(c + 1 < N_CHUNKS)
        def _pre():
            enqueue(c + 1)

        buf = lax.bitwise_and(c, 1)
        pltpu.make_async_copy(
            table_hbm.at[pl.ds(0, CIP)], rows_v.at[buf], sems.at[buf]
        ).wait()

        r0 = 4 * lax.bitwise_and(c, 1)
        for r in range(CR):
            base = r * L
            accs = [
                rows_v[buf, r0 + base, pl.ds(_OFFS[s], 16)]
                for s in range(len(_OFFS))
            ]
            for w in range(1, L):
                for s in range(len(_OFFS)):
                    accs[s] = accs[s] + rows_v[
                        buf, r0 + base + w, pl.ds(_OFFS[s], 16)
                    ]
            for s in range(len(_OFFS)):
                acc_v[pl.ds((CR * c + r) * H + _OFFS[s], 16)] = accs[s]
        return 0

    lax.fori_loop(0, N_CHUNKS, chunk_body, 0)

    pltpu.sync_copy(acc_v, out_hbm.at[pl.ds(wid * (BPW * H), BPW * H)])


def _sc_pool(idx_flat, table_pad):
    mesh = plsc.VectorSubcoreMesh(core_axis_name="c", subcore_axis_name="s")
    return pl.kernel(
        _sc_pool_body,
        out_type=jax.ShapeDtypeStruct((B * H,), jnp.float32),
        mesh=mesh,
        scratch_types=[
            pltpu.VMEM((BPW * L,), jnp.int32),
            pltpu.VMEM((2, CIP, HP), jnp.float32),
            pltpu.VMEM((BPW * H,), jnp.float32),
            pltpu.SemaphoreType.DMA((2,)),
        ],
    )(idx_flat, table_pad)


def _tc_matmul_body(x_ref, w_ref, b_ref, o_ref):
    x = x_ref[...] * (1.0 / L)
    o_ref[...] = (
        lax.dot_general(
            x, w_ref[...], (((1,), (1,)), ((), ())),
            preferred_element_type=jnp.float32,
        )
        + b_ref[...]
    )


def _tc_matmul(pooled, W, b2d):
    return pl.pallas_call(
        _tc_matmul_body,
        out_shape=jax.ShapeDtypeStruct((B, OUT), jnp.float32),
    )(pooled, W, b2d)


def kernel(keyword_tensor_list, word_embed, W, b):
    idx_flat = keyword_tensor_list.astype(jnp.int32).reshape(B * L)
    table_pad = _pad_table(word_embed)
    pooled = _sc_pool(idx_flat, table_pad).reshape(B, H)
    return _tc_matmul(pooled, W, b.reshape(1, OUT))
